# baseline (speedup 1.0000x reference)
import jax
import jax.numpy as jnp
from jax import lax
from jax.experimental import pallas as pl
from jax.experimental.pallas import tpu as pltpu
from jax.experimental.pallas import tpu_sc as plsc

N = 10000
E = 320000
D = 128
TAU = 0.2

NC = 2
NS = 16
NW = NC * NS
L = 16

NP = 10240
RPT = NP // NS
RPW = NP // NW
CH = 128
KC = 8
NB = 10
NCH = KC * NB
EPT = NCH * CH
EP = EPT * NW
NROW = NP // CH

f32 = jnp.float32
i32 = jnp.int32

_mesh = plsc.VectorSubcoreMesh(
    core_axis_name="c", subcore_axis_name="s", num_cores=NC, num_subcores=NS)
_cparams = pltpu.CompilerParams(needs_layout_passes=False)


def _rsqrt(v):
    i = plsc.bitcast(v, i32)
    y = plsc.bitcast(jnp.int32(0x5F3759DF) - (i >> 1), f32)
    for _ in range(4):
        y = y * (1.5 - 0.5 * v * y * y)
    return y


def _zeros16():
    return jnp.zeros((L,), f32)


def _worker():
    return lax.axis_index("c") * NS + lax.axis_index("s")


def _make_step_a(use_w):
    def body(*refs):
        if use_w:
            (yp_hbm, src_hbm, dst_hbm, w_hbm, zp_hbm,
             zsh, srcb, dstb, wb, gbuf0, gbuf1, srcS0, srcS1,
             dstS0, dstS1, wS0, wS1, zb, sem0, sem1, semS0, semS1) = refs
        else:
            (yp_hbm, src_hbm, dst_hbm, zp_hbm,
             zsh, srcb, dstb, wb, gbuf0, gbuf1, srcS0, srcS1,
             dstS0, dstS1, wS0, wS1, zb, sem0, sem1, semS0, semS1) = refs
            w_hbm = None
        gbuf = (gbuf0, gbuf1)
        srcS = (srcS0, srcS1)
        dstS = (dstS0, dstS1)
        wS = (wS0, wS1)
        sem = (sem0, sem1)
        sems = (semS0, semS1)
        c = lax.axis_index("c")
        s = lax.axis_index("s")

        zv = _zeros16()
        for r in range(L):
            for cc in range(D // L):
                zb[r, pl.ds(cc * L, L)] = zv

        def zrow(j, carry):
            pltpu.sync_copy(zb, zsh.at[pl.ds(s * RPT + j * L, L)])
            return carry
        lax.fori_loop(0, RPT // L, zrow, 0)
        plsc.subcore_barrier()

        wid = _worker()
        crow0 = wid * NCH

        def fire(i, pp):
            jj = i - (i // KC) * KC

            @pl.when(i >= 2)
            def _():
                pltpu.make_async_copy(gbuf[pp], zsh.at[dstS[pp]],
                                      sems[pp]).wait()

            @pl.when(jj == 0)
            def _():
                row0 = pl.multiple_of(crow0 + i, 8)
                pltpu.sync_copy(src_hbm.at[pl.ds(row0, KC)], srcb)
                pltpu.sync_copy(dst_hbm.at[pl.ds(row0, KC)], dstb)
                if use_w:
                    pltpu.sync_copy(w_hbm.at[pl.ds(row0, KC)], wb)

            for g in range(CH // L):
                sl = pl.ds(g * L, L)
                srcS[pp][sl] = srcb[jj, sl]
                dstS[pp][sl] = dstb[jj, sl]
                if use_w:
                    wS[pp][sl] = wb[jj, sl]
            pltpu.async_copy(yp_hbm.at[srcS[pp]], gbuf[pp], sem[pp])

        def retire(qq):
            pltpu.make_async_copy(yp_hbm.at[pl.ds(0, CH)], gbuf[qq],
                                  sem[qq]).wait()
            if use_w:
                def scale(g, carry3):
                    wvec = wS[qq][pl.ds(g * L, L)]
                    for r in range(L):
                        wv = wvec[r]
                        e = g * L + r
                        for cc in range(D // L):
                            sl = pl.ds(cc * L, L)
                            gbuf[qq][e, sl] = gbuf[qq][e, sl] * wv
                    return carry3
                lax.fori_loop(0, CH // L, scale, 0)
            pltpu.async_copy(gbuf[qq], zsh.at[dstS[qq]], sems[qq], add=True)

        def piter(i, carry):
            p = i & 1

            @pl.when((i < NCH) & (p == 0))
            def _():
                fire(i, 0)

            @pl.when((i < NCH) & (p == 1))
            def _():
                fire(i, 1)

            @pl.when((i > 0) & (p == 1))
            def _():
                retire(0)

            @pl.when((i > 0) & (p == 0))
            def _():
                retire(1)
            return carry
        lax.fori_loop(0, NCH + 1, piter, 0)
        pltpu.make_async_copy(gbuf[0], zsh.at[dstS[0]], sems[0]).wait()
        pltpu.make_async_copy(gbuf[1], zsh.at[dstS[1]], sems[1]).wait()
        plsc.subcore_barrier()

        def drow(j, carry):
            r0 = s * RPT + j * L
            pltpu.sync_copy(zsh.at[pl.ds(r0, L)], zb)
            pltpu.sync_copy(zb, zp_hbm.at[pl.ds(c * NP + r0, L)])
            return carry
        lax.fori_loop(0, RPT // L, drow, 0)

    scratch = [
        pltpu.VMEM_SHARED((NP, D), f32),
        pltpu.VMEM((KC, CH), i32),
        pltpu.VMEM((KC, CH), i32),
        pltpu.VMEM((KC, CH), f32),
        pltpu.VMEM((CH, D), f32),
        pltpu.VMEM((CH, D), f32),
        pltpu.VMEM((CH,), i32),
        pltpu.VMEM((CH,), i32),
        pltpu.VMEM((CH,), i32),
        pltpu.VMEM((CH,), i32),
        pltpu.VMEM((CH,), f32),
        pltpu.VMEM((CH,), f32),
        pltpu.VMEM((L, D), f32),
        pltpu.SemaphoreType.DMA,
        pltpu.SemaphoreType.DMA,
        pltpu.SemaphoreType.DMA,
        pltpu.SemaphoreType.DMA,
    ]
    out_type = jax.ShapeDtypeStruct((NC * NP, D), f32)
    return pl.kernel(body, out_type=out_type, mesh=_mesh,
                     scratch_types=scratch, compiler_params=_cparams)


_step_a_nw = _make_step_a(False)
_step_a_w = _make_step_a(True)


def _step_b_body(zp_hbm, ycur_hbm, ct_hbm, n1_hbm,
                 ynew_hbm, ypnew_hbm,
                 z0b, z1b, ybuf, cbuf, obuf, opbuf, n1t):
    wid = _worker()
    r0 = wid * RPW
    pltpu.sync_copy(n1_hbm.at[pl.ds(r0, RPW)], n1t)

    RB = 32

    def brow(j, carry):
        rr = r0 + j * RB
        pltpu.sync_copy(zp_hbm.at[pl.ds(rr, RB)], z0b)
        pltpu.sync_copy(zp_hbm.at[pl.ds(NP + rr, RB)], z1b)
        pltpu.sync_copy(ycur_hbm.at[pl.ds(rr, RB)], ybuf)
        pltpu.sync_copy(ct_hbm.at[pl.ds(rr, RB)], cbuf)
        for h in range(RB // L):
            n1v = n1t[pl.ds(j * RB + h * L, L)]
            for r in range(L):
                nv = n1v[r]
                e = h * L + r
                for cc in range(D // L):
                    sl = pl.ds(cc * L, L)
                    out = 0.5 * ybuf[e, sl] \
                        + (0.5 * nv) * (z0b[e, sl] + z1b[e, sl]) \
                        + cbuf[e, sl]
                    obuf[e, sl] = out
                    opbuf[e, sl] = nv * out
        pltpu.sync_copy(obuf, ynew_hbm.at[pl.ds(rr, RB)])
        pltpu.sync_copy(opbuf, ypnew_hbm.at[pl.ds(rr, RB)])
        return carry
    lax.fori_loop(0, RPW // RB, brow, 0)


_step_b = pl.kernel(
    _step_b_body,
    out_type=(jax.ShapeDtypeStruct((NP, D), f32),
              jax.ShapeDtypeStruct((NP, D), f32)),
    mesh=_mesh,
    compiler_params=_cparams,
    scratch_types=[
        pltpu.VMEM((32, D), f32),
        pltpu.VMEM((32, D), f32),
        pltpu.VMEM((32, D), f32),
        pltpu.VMEM((32, D), f32),
        pltpu.VMEM((32, D), f32),
        pltpu.VMEM((32, D), f32),
        pltpu.VMEM((RPW,), f32),
    ])


def _make_prep_a(with_w):
    def body(*refs):
        if with_w:
            (src_hbm, dst_hbm, hn_hbm, pd_hbm,
             degp_hbm, w_hbm,
             degsh, degloc1, degloc, riB, srcb, dstb, wb, pdb, hnf,
             zbd) = refs
        else:
            (dst_hbm,
             degp_hbm,
             degsh, degloc1, degloc, riB, srcb, dstb, wb, pdb, hnf,
             zbd) = refs
        c = lax.axis_index("c")
        s = lax.axis_index("s")
        zv = _zeros16()
        it = lax.iota(i32, L)

        for j in range(8):
            for g in range(CH // L):
                zbd[j, pl.ds(g * L, L)] = zv

        @pl.when(s < NROW // 8)
        def _():
            pltpu.sync_copy(zbd, degsh.at[pl.ds(s * 8, 8)])

        def zloc(j, carry):
            degloc1[pl.ds(j * L, L)] = zv
            return carry
        lax.fori_loop(0, NP // L, zloc, 0)

        for g in range(NROW // L):
            riB[pl.ds(g * L, L)] = it + g * L
        plsc.subcore_barrier()

        if with_w:
            pltpu.sync_copy(hn_hbm.at[pl.ds(0, NP)], hnf)

        wid = _worker()
        bbase = wid * NB

        def block(i, carry):
            row0 = (bbase + i) * KC
            pltpu.sync_copy(dst_hbm.at[pl.ds(row0, KC)], dstb)
            if with_w:
                pltpu.sync_copy(src_hbm.at[pl.ds(row0, KC)], srcb)
                pltpu.sync_copy(pd_hbm.at[pl.ds(row0, KC)], pdb)

            def chunk(j, carry2):
                for g in range(CH // L):
                    sl = pl.ds(g * L, L)
                    dstv = dstb[j, sl]
                    if with_w:
                        srcv = srcb[j, sl]
                        hs = plsc.load_gather(hnf, [srcv])
                        hd = plsc.load_gather(hnf, [dstv])
                        wr = hs + hd - 2.0 * pdb[j, sl]
                        m = jnp.maximum(wr, 0.0) + 1e-7
                        wv = jnp.minimum(_rsqrt(m), 1.0 / TAU) + 1e-9
                        wb[j, sl] = wv
                    else:
                        wv = zv + 1.0
                    plsc.addupdate_scatter(degloc1, [dstv], wv)
                return carry2
            lax.fori_loop(0, KC, chunk, 0)
            if with_w:
                pltpu.sync_copy(wb, w_hbm.at[pl.ds(row0, KC)])
            return carry
        lax.fori_loop(0, NB, block, 0)

        def repack(j, carry):
            for g in range(CH // L):
                degloc[j, pl.ds(g * L, L)] = \
                    degloc1[pl.ds(j * CH + g * L, L)]
            return carry
        lax.fori_loop(0, NROW, repack, 0)

        pltpu.sync_copy(degloc, degsh.at[riB], add=True)
        plsc.subcore_barrier()

        @pl.when(s < NROW // 8)
        def _():
            pltpu.sync_copy(degsh.at[pl.ds(s * 8, 8)], zbd)
            pltpu.sync_copy(zbd, degp_hbm.at[pl.ds(c * NROW + s * 8, 8)])

    scratch = [
        pltpu.VMEM_SHARED((NROW, CH), f32),
        pltpu.VMEM((NP,), f32),
        pltpu.VMEM((NROW, CH), f32),
        pltpu.VMEM((NROW,), i32),
        pltpu.VMEM((KC, CH), i32),
        pltpu.VMEM((KC, CH), i32),
        pltpu.VMEM((KC, CH), f32),
        pltpu.VMEM((KC, CH), f32),
        pltpu.VMEM((NP,), f32),
        pltpu.VMEM((8, CH), f32),
    ]
    outs = [jax.ShapeDtypeStruct((NC * NROW, CH), f32)]
    if with_w:
        outs.append(jax.ShapeDtypeStruct((EP // CH, CH), f32))
    out_type = outs[0] if not with_w else tuple(outs)
    return pl.kernel(body, out_type=out_type, mesh=_mesh,
                     scratch_types=scratch, compiler_params=_cparams)


_prep_a1 = _make_prep_a(False)
_prep_a2 = _make_prep_a(True)


def _prep_b_body(x_hbm, ysrc_hbm, degp_hbm,
                 n1_hbm, ct_hbm, yp_hbm,
                 degb, xbuf, ybuf, ctb, ypb, n1t):
    s = lax.axis_index("s")

    pltpu.sync_copy(degp_hbm, degb)

    def nrow(j, carry):
        r0 = s * RPT + j * L
        dr = s * (NROW // NS) + (j >> 3)
        dsl = pl.ds((j & 7) * L, L)
        degv = degb[dr, dsl] + degb[NROW + dr, dsl]
        n1v = _rsqrt(degv)
        dvv = 0.5 * (n1v * n1v)
        n1t[pl.ds(j * L, L)] = n1v
        pltpu.sync_copy(x_hbm.at[pl.ds(r0, L)], xbuf)
        pltpu.sync_copy(ysrc_hbm.at[pl.ds(r0, L)], ybuf)
        for r in range(L):
            nv = n1v[r]
            dv = dvv[r]
            for cc in range(D // L):
                sl = pl.ds(cc * L, L)
                ctb[r, sl] = dv * xbuf[r, sl]
                ypb[r, sl] = nv * ybuf[r, sl]
        pltpu.sync_copy(ctb, ct_hbm.at[pl.ds(r0, L)])
        pltpu.sync_copy(ypb, yp_hbm.at[pl.ds(r0, L)])
        return carry
    lax.fori_loop(0, RPT // L, nrow, 0)
    pltpu.sync_copy(n1t, n1_hbm.at[pl.ds(s * RPT, RPT)])


_prep_b = pl.kernel(
    _prep_b_body,
    out_type=(jax.ShapeDtypeStruct((NP,), f32),
              jax.ShapeDtypeStruct((NP, D), f32),
              jax.ShapeDtypeStruct((NP, D), f32)),
    mesh=_mesh,
    compiler_params=_cparams,
    scratch_types=[
        pltpu.VMEM((NC * NROW, CH), f32),
        pltpu.VMEM((L, D), f32),
        pltpu.VMEM((L, D), f32),
        pltpu.VMEM((L, D), f32),
        pltpu.VMEM((L, D), f32),
        pltpu.VMEM((RPT,), f32),
    ])


def _lane_sum(tbuf, it):
    acc = _zeros16()
    for l in range(L):
        col = jnp.zeros((L,), i32) + l
        acc = acc + plsc.load_gather(tbuf, [it, col])
    return acc


def _attn_body(ys_hbm, src_hbm, dst_hbm, hn_hbm, pd_hbm,
               srcb, dstb, ga0, ga1, gb0, gb1, srcS0, srcS1, dstS0, dstS1,
               ybuf, tbuf, hnt, pdb, sma0, sma1, smb0, smb1):
    it = lax.iota(i32, L)
    wid = _worker()

    r0 = wid * RPW

    def hrow(j, carry):
        pltpu.sync_copy(ys_hbm.at[pl.ds(r0 + j * L, L)], ybuf)
        for r in range(L):
            acc = _zeros16()
            for sl in range(D // L):
                v = ybuf[r, pl.ds(sl * L, L)]
                acc = acc + v * v
            tbuf[r, pl.ds(0, L)] = acc
        hnt[pl.ds(j * L, L)] = _lane_sum(tbuf, it)
        return carry
    lax.fori_loop(0, RPW // L, hrow, 0)
    pltpu.sync_copy(hnt, hn_hbm.at[pl.ds(r0, RPW)])

    crow0 = wid * NCH
    ga = (ga0, ga1)
    gb = (gb0, gb1)
    srcS = (srcS0, srcS1)
    dstS = (dstS0, dstS1)
    sma = (sma0, sma1)
    smb = (smb0, smb1)

    def fire(i, pp):
        jj = i - (i // KC) * KC

        @pl.when(jj == 0)
        def _():
            row0 = pl.multiple_of(crow0 + i, 8)
            pltpu.sync_copy(src_hbm.at[pl.ds(row0, KC)], srcb)
            pltpu.sync_copy(dst_hbm.at[pl.ds(row0, KC)], dstb)
        for g in range(CH // L):
            sl = pl.ds(g * L, L)
            srcS[pp][sl] = srcb[jj, sl]
            dstS[pp][sl] = dstb[jj, sl]
        pltpu.async_copy(ys_hbm.at[srcS[pp]], ga[pp], sma[pp])
        pltpu.async_copy(ys_hbm.at[dstS[pp]], gb[pp], smb[pp])

    def retire(i, qq):
        iq = i - 1
        jq = iq - (iq // KC) * KC
        pltpu.make_async_copy(ys_hbm.at[pl.ds(0, CH)], ga[qq],
                              sma[qq]).wait()
        pltpu.make_async_copy(ys_hbm.at[pl.ds(0, CH)], gb[qq],
                              smb[qq]).wait()

        def grp(g, carry3):
            for r in range(L):
                e = g * L + r
                acc = _zeros16()
                for sl in range(D // L):
                    csl = pl.ds(sl * L, L)
                    acc = acc + ga[qq][e, csl] * gb[qq][e, csl]
                tbuf[r, pl.ds(0, L)] = acc
            pdb[jq, pl.ds(g * L, L)] = _lane_sum(tbuf, it)
            return carry3
        lax.fori_loop(0, CH // L, grp, 0)

        @pl.when(jq == KC - 1)
        def _():
            row0 = pl.multiple_of(crow0 + iq - (KC - 1), 8)
            pltpu.sync_copy(pdb, pd_hbm.at[pl.ds(row0, KC)])

    def piter(i, carry):
        p = i & 1

        @pl.when((i < NCH) & (p == 0))
        def _():
            fire(i, 0)

        @pl.when((i < NCH) & (p == 1))
        def _():
            fire(i, 1)

        @pl.when((i > 0) & (p == 1))
        def _():
            retire(i, 0)

        @pl.when((i > 0) & (p == 0))
        def _():
            retire(i, 1)
        return carry
    lax.fori_loop(0, NCH + 1, piter, 0)


_attn = pl.kernel(
    _attn_body,
    out_type=(jax.ShapeDtypeStruct((NP,), f32),
              jax.ShapeDtypeStruct((EP // CH, CH), f32)),
    mesh=_mesh,
    compiler_params=_cparams,
    scratch_types=[
        pltpu.VMEM((KC, CH), i32),
        pltpu.VMEM((KC, CH), i32),
        pltpu.VMEM((CH, D), f32),
        pltpu.VMEM((CH, D), f32),
        pltpu.VMEM((CH, D), f32),
        pltpu.VMEM((CH, D), f32),
        pltpu.VMEM((CH,), i32),
        pltpu.VMEM((CH,), i32),
        pltpu.VMEM((CH,), i32),
        pltpu.VMEM((CH,), i32),
        pltpu.VMEM((L, D), f32),
        pltpu.VMEM((L, L), f32),
        pltpu.VMEM((RPW,), f32),
        pltpu.VMEM((KC, CH), f32),
        pltpu.SemaphoreType.DMA,
        pltpu.SemaphoreType.DMA,
        pltpu.SemaphoreType.DMA,
        pltpu.SemaphoreType.DMA,
    ])


def kernel(x, edge_index):
    src = edge_index[0].astype(i32)
    dst = edge_index[1].astype(i32)
    srcp = jnp.zeros((EP,), i32).at[:E].set(src).reshape(EP // CH, CH)
    dstp = jnp.full((EP,), NP - 1, i32).at[:E].set(dst).reshape(EP // CH, CH)

    xs = jnp.zeros((NP, D), f32).at[:N].set(x)

    degp = _prep_a1(dstp)
    n1, ct, yp = _prep_b(xs, xs, degp)
    ys = xs
    for _ in range(4):
        zp = _step_a_nw(yp, srcp, dstp)
        ys, yp = _step_b(zp, ys, ct, n1)
    hn, pd = _attn(ys, srcp, dstp)
    degp, w = _prep_a2(srcp, dstp, hn, pd)
    n1, ct, yp = _prep_b(xs, ys, degp)
    for _ in range(4):
        zp = _step_a_w(yp, srcp, dstp, w)
        ys, yp = _step_b(zp, ys, ct, n1)
    return ys[:N]


# speedup vs baseline: 1.0145x; 1.0145x over previous
import jax
import jax.numpy as jnp
from jax import lax
from jax.experimental import pallas as pl
from jax.experimental.pallas import tpu as pltpu
from jax.experimental.pallas import tpu_sc as plsc

N = 10000
E = 320000
D = 128
TAU = 0.2

NC = 2
NS = 16
NW = NC * NS
L = 16

NP = 10240
RPT = NP // NS
RPW = NP // NW
CH = 128
KC = 8
NB = 10
NCH = KC * NB
EPT = NCH * CH
EP = EPT * NW
NROW = NP // CH

f32 = jnp.float32
i32 = jnp.int32

_mesh = plsc.VectorSubcoreMesh(
    core_axis_name="c", subcore_axis_name="s", num_cores=NC, num_subcores=NS)
_cparams = pltpu.CompilerParams(needs_layout_passes=False)


def _rsqrt(v):
    i = plsc.bitcast(v, i32)
    y = plsc.bitcast(jnp.int32(0x5F3759DF) - (i >> 1), f32)
    for _ in range(4):
        y = y * (1.5 - 0.5 * v * y * y)
    return y


def _zeros16():
    return jnp.zeros((L,), f32)


def _worker():
    return lax.axis_index("c") * NS + lax.axis_index("s")


def _make_step_a(use_w):
    def body(*refs):
        if use_w:
            (yp_hbm, src_hbm, dst_hbm, w_hbm, zp_hbm,
             zsh, srcb, dstb, wb, gbuf0, gbuf1, srcS0, srcS1,
             dstS0, dstS1, wS0, wS1, zb, sem0, sem1, semS0, semS1) = refs
        else:
            (yp_hbm, src_hbm, dst_hbm, zp_hbm,
             zsh, srcb, dstb, wb, gbuf0, gbuf1, srcS0, srcS1,
             dstS0, dstS1, wS0, wS1, zb, sem0, sem1, semS0, semS1) = refs
            w_hbm = None
        gbuf = (gbuf0, gbuf1)
        srcS = (srcS0, srcS1)
        dstS = (dstS0, dstS1)
        wS = (wS0, wS1)
        sem = (sem0, sem1)
        sems = (semS0, semS1)
        c = lax.axis_index("c")
        s = lax.axis_index("s")

        zv = _zeros16()
        for r in range(L):
            for cc in range(D // L):
                zb[r, pl.ds(cc * L, L)] = zv

        def zrow(j, carry):
            pltpu.sync_copy(zb, zsh.at[pl.ds(s * RPT + j * L, L)])
            return carry
        lax.fori_loop(0, RPT // L, zrow, 0)
        plsc.subcore_barrier()

        wid = _worker()
        crow0 = wid * NCH

        def fire(i, pp):
            jj = i - (i // KC) * KC

            @pl.when(i >= 2)
            def _():
                pltpu.make_async_copy(gbuf[pp], zsh.at[dstS[pp]],
                                      sems[pp]).wait()

            @pl.when(jj == 0)
            def _():
                row0 = pl.multiple_of(crow0 + i, 8)
                pltpu.sync_copy(src_hbm.at[pl.ds(row0, KC)], srcb)
                pltpu.sync_copy(dst_hbm.at[pl.ds(row0, KC)], dstb)
                if use_w:
                    pltpu.sync_copy(w_hbm.at[pl.ds(row0, KC)], wb)

            for g in range(CH // L):
                sl = pl.ds(g * L, L)
                srcS[pp][sl] = srcb[jj, sl]
                dstS[pp][sl] = dstb[jj, sl]
                if use_w:
                    wS[pp][sl] = wb[jj, sl]
            pltpu.async_copy(yp_hbm.at[srcS[pp]], gbuf[pp], sem[pp])

        def retire(qq):
            pltpu.make_async_copy(yp_hbm.at[pl.ds(0, CH)], gbuf[qq],
                                  sem[qq]).wait()
            if use_w:
                def scale(g, carry3):
                    wvec = wS[qq][pl.ds(g * L, L)]
                    for r in range(L):
                        wv = wvec[r]
                        e = g * L + r
                        for cc in range(D // L):
                            sl = pl.ds(cc * L, L)
                            gbuf[qq][e, sl] = gbuf[qq][e, sl] * wv
                    return carry3
                lax.fori_loop(0, CH // L, scale, 0)
            pltpu.async_copy(gbuf[qq], zsh.at[dstS[qq]], sems[qq], add=True)

        def piter(i, carry):
            p = i & 1

            @pl.when((i < NCH) & (p == 0))
            def _():
                fire(i, 0)

            @pl.when((i < NCH) & (p == 1))
            def _():
                fire(i, 1)

            @pl.when((i > 0) & (p == 1))
            def _():
                retire(0)

            @pl.when((i > 0) & (p == 0))
            def _():
                retire(1)
            return carry
        lax.fori_loop(0, NCH + 1, piter, 0)
        pltpu.make_async_copy(gbuf[0], zsh.at[dstS[0]], sems[0]).wait()
        pltpu.make_async_copy(gbuf[1], zsh.at[dstS[1]], sems[1]).wait()
        plsc.subcore_barrier()

        def drow(j, carry):
            r0 = s * RPT + j * L
            pltpu.sync_copy(zsh.at[pl.ds(r0, L)], zb)
            pltpu.sync_copy(zb, zp_hbm.at[pl.ds(c * NP + r0, L)])
            return carry
        lax.fori_loop(0, RPT // L, drow, 0)

    scratch = [
        pltpu.VMEM_SHARED((NP, D), f32),
        pltpu.VMEM((KC, CH), i32),
        pltpu.VMEM((KC, CH), i32),
        pltpu.VMEM((KC, CH), f32),
        pltpu.VMEM((CH, D), f32),
        pltpu.VMEM((CH, D), f32),
        pltpu.VMEM((CH,), i32),
        pltpu.VMEM((CH,), i32),
        pltpu.VMEM((CH,), i32),
        pltpu.VMEM((CH,), i32),
        pltpu.VMEM((CH,), f32),
        pltpu.VMEM((CH,), f32),
        pltpu.VMEM((L, D), f32),
        pltpu.SemaphoreType.DMA,
        pltpu.SemaphoreType.DMA,
        pltpu.SemaphoreType.DMA,
        pltpu.SemaphoreType.DMA,
    ]
    out_type = jax.ShapeDtypeStruct((NC * NP, D), f32)
    return pl.kernel(body, out_type=out_type, mesh=_mesh,
                     scratch_types=scratch, compiler_params=_cparams)


_step_a_w = _make_step_a(True)


def _step_b_body(zp_hbm, ycur_hbm, ct_hbm, n1_hbm,
                 ynew_hbm, ypnew_hbm,
                 z0b, z1b, ybuf, cbuf, obuf, opbuf, n1t):
    wid = _worker()
    r0 = wid * RPW
    pltpu.sync_copy(n1_hbm.at[pl.ds(r0, RPW)], n1t)

    RB = 32

    def brow(j, carry):
        rr = r0 + j * RB
        pltpu.sync_copy(zp_hbm.at[pl.ds(rr, RB)], z0b)
        pltpu.sync_copy(zp_hbm.at[pl.ds(NP + rr, RB)], z1b)
        pltpu.sync_copy(ycur_hbm.at[pl.ds(rr, RB)], ybuf)
        pltpu.sync_copy(ct_hbm.at[pl.ds(rr, RB)], cbuf)
        for h in range(RB // L):
            n1v = n1t[pl.ds(j * RB + h * L, L)]
            for r in range(L):
                nv = n1v[r]
                e = h * L + r
                for cc in range(D // L):
                    sl = pl.ds(cc * L, L)
                    out = 0.5 * ybuf[e, sl] \
                        + (0.5 * nv) * (z0b[e, sl] + z1b[e, sl]) \
                        + cbuf[e, sl]
                    obuf[e, sl] = out
                    opbuf[e, sl] = nv * out
        pltpu.sync_copy(obuf, ynew_hbm.at[pl.ds(rr, RB)])
        pltpu.sync_copy(opbuf, ypnew_hbm.at[pl.ds(rr, RB)])
        return carry
    lax.fori_loop(0, RPW // RB, brow, 0)


_step_b = pl.kernel(
    _step_b_body,
    out_type=(jax.ShapeDtypeStruct((NP, D), f32),
              jax.ShapeDtypeStruct((NP, D), f32)),
    mesh=_mesh,
    compiler_params=_cparams,
    scratch_types=[
        pltpu.VMEM((32, D), f32),
        pltpu.VMEM((32, D), f32),
        pltpu.VMEM((32, D), f32),
        pltpu.VMEM((32, D), f32),
        pltpu.VMEM((32, D), f32),
        pltpu.VMEM((32, D), f32),
        pltpu.VMEM((RPW,), f32),
    ])


def _make_prep_a(with_w):
    def body(*refs):
        if with_w:
            (src_hbm, dst_hbm, hn_hbm, pd_hbm,
             degp_hbm, w_hbm,
             degsh, degloc1, degloc, riB, srcb, dstb, wb, pdb, hnf,
             zbd) = refs
        else:
            (dst_hbm,
             degp_hbm,
             degsh, degloc1, degloc, riB, srcb, dstb, wb, pdb, hnf,
             zbd) = refs
        c = lax.axis_index("c")
        s = lax.axis_index("s")
        zv = _zeros16()
        it = lax.iota(i32, L)

        for j in range(8):
            for g in range(CH // L):
                zbd[j, pl.ds(g * L, L)] = zv

        @pl.when(s < NROW // 8)
        def _():
            pltpu.sync_copy(zbd, degsh.at[pl.ds(s * 8, 8)])

        def zloc(j, carry):
            degloc1[pl.ds(j * L, L)] = zv
            return carry
        lax.fori_loop(0, NP // L, zloc, 0)

        for g in range(NROW // L):
            riB[pl.ds(g * L, L)] = it + g * L
        plsc.subcore_barrier()

        if with_w:
            pltpu.sync_copy(hn_hbm.at[pl.ds(0, NP)], hnf)

        wid = _worker()
        bbase = wid * NB

        def block(i, carry):
            row0 = (bbase + i) * KC
            pltpu.sync_copy(dst_hbm.at[pl.ds(row0, KC)], dstb)
            if with_w:
                pltpu.sync_copy(src_hbm.at[pl.ds(row0, KC)], srcb)
                pltpu.sync_copy(pd_hbm.at[pl.ds(row0, KC)], pdb)

            def chunk(j, carry2):
                for g in range(CH // L):
                    sl = pl.ds(g * L, L)
                    dstv = dstb[j, sl]
                    if with_w:
                        srcv = srcb[j, sl]
                        hs = plsc.load_gather(hnf, [srcv])
                        hd = plsc.load_gather(hnf, [dstv])
                        wr = hs + hd - 2.0 * pdb[j, sl]
                        m = jnp.maximum(wr, 0.0) + 1e-7
                        wv = jnp.minimum(_rsqrt(m), 1.0 / TAU) + 1e-9
                        wb[j, sl] = wv
                    else:
                        wv = zv + 1.0
                    plsc.addupdate_scatter(degloc1, [dstv], wv)
                return carry2
            lax.fori_loop(0, KC, chunk, 0)
            if with_w:
                pltpu.sync_copy(wb, w_hbm.at[pl.ds(row0, KC)])
            return carry
        lax.fori_loop(0, NB, block, 0)

        def repack(j, carry):
            for g in range(CH // L):
                degloc[j, pl.ds(g * L, L)] = \
                    degloc1[pl.ds(j * CH + g * L, L)]
            return carry
        lax.fori_loop(0, NROW, repack, 0)

        pltpu.sync_copy(degloc, degsh.at[riB], add=True)
        plsc.subcore_barrier()

        @pl.when(s < NROW // 8)
        def _():
            pltpu.sync_copy(degsh.at[pl.ds(s * 8, 8)], zbd)
            pltpu.sync_copy(zbd, degp_hbm.at[pl.ds(c * NROW + s * 8, 8)])

    scratch = [
        pltpu.VMEM_SHARED((NROW, CH), f32),
        pltpu.VMEM((NP,), f32),
        pltpu.VMEM((NROW, CH), f32),
        pltpu.VMEM((NROW,), i32),
        pltpu.VMEM((KC, CH), i32),
        pltpu.VMEM((KC, CH), i32),
        pltpu.VMEM((KC, CH), f32),
        pltpu.VMEM((KC, CH), f32),
        pltpu.VMEM((NP,), f32),
        pltpu.VMEM((8, CH), f32),
    ]
    outs = [jax.ShapeDtypeStruct((NC * NROW, CH), f32)]
    if with_w:
        outs.append(jax.ShapeDtypeStruct((EP // CH, CH), f32))
    out_type = outs[0] if not with_w else tuple(outs)
    return pl.kernel(body, out_type=out_type, mesh=_mesh,
                     scratch_types=scratch, compiler_params=_cparams)


_prep_a1 = _make_prep_a(False)
_prep_a2 = _make_prep_a(True)


def _prep_b_body(x_hbm, ysrc_hbm, degp_hbm,
                 n1_hbm, ct_hbm, yp_hbm,
                 degb, xbuf, ybuf, ctb, ypb, n1t):
    s = lax.axis_index("s")

    pltpu.sync_copy(degp_hbm, degb)

    RB = 32

    def nrow(j, carry):
        r0 = s * RPT + j * RB
        pltpu.sync_copy(x_hbm.at[pl.ds(r0, RB)], xbuf)
        pltpu.sync_copy(ysrc_hbm.at[pl.ds(r0, RB)], ybuf)
        for h in range(RB // L):
            jj = j * (RB // L) + h
            dr = s * (NROW // NS) + (jj >> 3)
            dsl = pl.ds((jj & 7) * L, L)
            degv = degb[dr, dsl] + degb[NROW + dr, dsl]
            n1v = _rsqrt(degv)
            dvv = 0.5 * (n1v * n1v)
            n1t[pl.ds(jj * L, L)] = n1v
            for r in range(L):
                nv = n1v[r]
                dv = dvv[r]
                e = h * L + r
                for cc in range(D // L):
                    sl = pl.ds(cc * L, L)
                    ctb[e, sl] = dv * xbuf[e, sl]
                    ypb[e, sl] = nv * ybuf[e, sl]
        pltpu.sync_copy(ctb, ct_hbm.at[pl.ds(r0, RB)])
        pltpu.sync_copy(ypb, yp_hbm.at[pl.ds(r0, RB)])
        return carry
    lax.fori_loop(0, RPT // RB, nrow, 0)
    pltpu.sync_copy(n1t, n1_hbm.at[pl.ds(s * RPT, RPT)])


_prep_b = pl.kernel(
    _prep_b_body,
    out_type=(jax.ShapeDtypeStruct((NP,), f32),
              jax.ShapeDtypeStruct((NP, D), f32),
              jax.ShapeDtypeStruct((NP, D), f32)),
    mesh=_mesh,
    compiler_params=_cparams,
    scratch_types=[
        pltpu.VMEM((NC * NROW, CH), f32),
        pltpu.VMEM((32, D), f32),
        pltpu.VMEM((32, D), f32),
        pltpu.VMEM((32, D), f32),
        pltpu.VMEM((32, D), f32),
        pltpu.VMEM((RPT,), f32),
    ])


def _lane_sum(tbuf, it):
    acc = _zeros16()
    for l in range(L):
        col = jnp.zeros((L,), i32) + l
        acc = acc + plsc.load_gather(tbuf, [it, col])
    return acc


def _attn_body(ys_hbm, src_hbm, dst_hbm, hn_hbm, pd_hbm,
               srcb, dstb, ga0, ga1, gb0, gb1, srcS0, srcS1, dstS0, dstS1,
               ybuf, tbuf, hnt, pdb, sma0, sma1, smb0, smb1):
    it = lax.iota(i32, L)
    wid = _worker()

    r0 = wid * RPW

    def hrow(j, carry):
        pltpu.sync_copy(ys_hbm.at[pl.ds(r0 + j * L, L)], ybuf)
        for r in range(L):
            acc = _zeros16()
            for sl in range(D // L):
                v = ybuf[r, pl.ds(sl * L, L)]
                acc = acc + v * v
            tbuf[r, pl.ds(0, L)] = acc
        hnt[pl.ds(j * L, L)] = _lane_sum(tbuf, it)
        return carry
    lax.fori_loop(0, RPW // L, hrow, 0)
    pltpu.sync_copy(hnt, hn_hbm.at[pl.ds(r0, RPW)])

    crow0 = wid * NCH
    ga = (ga0, ga1)
    gb = (gb0, gb1)
    srcS = (srcS0, srcS1)
    dstS = (dstS0, dstS1)
    sma = (sma0, sma1)
    smb = (smb0, smb1)

    def fire(i, pp):
        jj = i - (i // KC) * KC

        @pl.when(jj == 0)
        def _():
            row0 = pl.multiple_of(crow0 + i, 8)
            pltpu.sync_copy(src_hbm.at[pl.ds(row0, KC)], srcb)
            pltpu.sync_copy(dst_hbm.at[pl.ds(row0, KC)], dstb)
        for g in range(CH // L):
            sl = pl.ds(g * L, L)
            srcS[pp][sl] = srcb[jj, sl]
            dstS[pp][sl] = dstb[jj, sl]
        pltpu.async_copy(ys_hbm.at[srcS[pp]], ga[pp], sma[pp])
        pltpu.async_copy(ys_hbm.at[dstS[pp]], gb[pp], smb[pp])

    def retire(i, qq):
        iq = i - 1
        jq = iq - (iq // KC) * KC
        pltpu.make_async_copy(ys_hbm.at[pl.ds(0, CH)], ga[qq],
                              sma[qq]).wait()
        pltpu.make_async_copy(ys_hbm.at[pl.ds(0, CH)], gb[qq],
                              smb[qq]).wait()

        def grp(g, carry3):
            for r in range(L):
                e = g * L + r
                acc = _zeros16()
                for sl in range(D // L):
                    csl = pl.ds(sl * L, L)
                    acc = acc + ga[qq][e, csl] * gb[qq][e, csl]
                tbuf[r, pl.ds(0, L)] = acc
            pdb[jq, pl.ds(g * L, L)] = _lane_sum(tbuf, it)
            return carry3
        lax.fori_loop(0, CH // L, grp, 0)

        @pl.when(jq == KC - 1)
        def _():
            row0 = pl.multiple_of(crow0 + iq - (KC - 1), 8)
            pltpu.sync_copy(pdb, pd_hbm.at[pl.ds(row0, KC)])

    def piter(i, carry):
        p = i & 1

        @pl.when((i < NCH) & (p == 0))
        def _():
            fire(i, 0)

        @pl.when((i < NCH) & (p == 1))
        def _():
            fire(i, 1)

        @pl.when((i > 0) & (p == 1))
        def _():
            retire(i, 0)

        @pl.when((i > 0) & (p == 0))
        def _():
            retire(i, 1)
        return carry
    lax.fori_loop(0, NCH + 1, piter, 0)


_attn = pl.kernel(
    _attn_body,
    out_type=(jax.ShapeDtypeStruct((NP,), f32),
              jax.ShapeDtypeStruct((EP // CH, CH), f32)),
    mesh=_mesh,
    compiler_params=_cparams,
    scratch_types=[
        pltpu.VMEM((KC, CH), i32),
        pltpu.VMEM((KC, CH), i32),
        pltpu.VMEM((CH, D), f32),
        pltpu.VMEM((CH, D), f32),
        pltpu.VMEM((CH, D), f32),
        pltpu.VMEM((CH, D), f32),
        pltpu.VMEM((CH,), i32),
        pltpu.VMEM((CH,), i32),
        pltpu.VMEM((CH,), i32),
        pltpu.VMEM((CH,), i32),
        pltpu.VMEM((L, D), f32),
        pltpu.VMEM((L, L), f32),
        pltpu.VMEM((RPW,), f32),
        pltpu.VMEM((KC, CH), f32),
        pltpu.SemaphoreType.DMA,
        pltpu.SemaphoreType.DMA,
        pltpu.SemaphoreType.DMA,
        pltpu.SemaphoreType.DMA,
    ])


def kernel(x, edge_index):
    src = edge_index[0].astype(i32)
    dst = edge_index[1].astype(i32)
    srcp = jnp.zeros((EP,), i32).at[:E].set(src).reshape(EP // CH, CH)
    dstp = jnp.full((EP,), NP - 1, i32).at[:E].set(dst).reshape(EP // CH, CH)

    xs = jnp.zeros((NP, D), f32).at[:N].set(x)

    degp = _prep_a1(dstp)
    n1, ct, yp = _prep_b(xs, xs, degp)
    ys = xs
    w1 = jnp.ones((EP // CH, CH), f32)
    for _ in range(4):
        zp = _step_a_w(yp, srcp, dstp, w1)
        ys, yp = _step_b(zp, ys, ct, n1)
    hn, pd = _attn(ys, srcp, dstp)
    degp, w = _prep_a2(srcp, dstp, hn, pd)
    n1, ct, yp = _prep_b(xs, ys, degp)
    for _ in range(4):
        zp = _step_a_w(yp, srcp, dstp, w)
        ys, yp = _step_b(zp, ys, ct, n1)
    return ys[:N]


# speedup vs baseline: 1.0290x; 1.0143x over previous
import jax
import jax.numpy as jnp
from jax import lax
from jax.experimental import pallas as pl
from jax.experimental.pallas import tpu as pltpu
from jax.experimental.pallas import tpu_sc as plsc

N = 10000
E = 320000
D = 128
TAU = 0.2

NC = 2
NS = 16
NW = NC * NS
L = 16

NP = 10240
RPT = NP // NS
RPW = NP // NW
CH = 128
KC = 8
NB = 10
NCH = KC * NB
EPT = NCH * CH
EP = EPT * NW
NROW = NP // CH

f32 = jnp.float32
i32 = jnp.int32

_mesh = plsc.VectorSubcoreMesh(
    core_axis_name="c", subcore_axis_name="s", num_cores=NC, num_subcores=NS)
_cparams = pltpu.CompilerParams(needs_layout_passes=False)


def _rsqrt(v):
    i = plsc.bitcast(v, i32)
    y = plsc.bitcast(jnp.int32(0x5F3759DF) - (i >> 1), f32)
    for _ in range(4):
        y = y * (1.5 - 0.5 * v * y * y)
    return y


def _zeros16():
    return jnp.zeros((L,), f32)


def _worker():
    return lax.axis_index("c") * NS + lax.axis_index("s")


def _make_step_a(use_w):
    def body(*refs):
        if use_w:
            (yp_hbm, src_hbm, dst_hbm, w_hbm, zp_hbm,
             zsh, srcb, dstb, wb, gbuf0, gbuf1, srcS0, srcS1,
             dstS0, dstS1, wS0, wS1, zb, sem0, sem1, semS0, semS1) = refs
        else:
            (yp_hbm, src_hbm, dst_hbm, zp_hbm,
             zsh, srcb, dstb, wb, gbuf0, gbuf1, srcS0, srcS1,
             dstS0, dstS1, wS0, wS1, zb, sem0, sem1, semS0, semS1) = refs
            w_hbm = None
        gbuf = (gbuf0, gbuf1)
        srcS = (srcS0, srcS1)
        dstS = (dstS0, dstS1)
        wS = (wS0, wS1)
        sem = (sem0, sem1)
        sems = (semS0, semS1)
        c = lax.axis_index("c")
        s = lax.axis_index("s")

        ZR = 64
        zv = _zeros16()

        def zfill(r, carry):
            for cc in range(D // L):
                zb[r, pl.ds(cc * L, L)] = zv
            return carry
        lax.fori_loop(0, ZR, zfill, 0)

        def zrow(j, carry):
            pltpu.sync_copy(zb, zsh.at[pl.ds(s * RPT + j * ZR, ZR)])
            return carry
        lax.fori_loop(0, RPT // ZR, zrow, 0)
        plsc.subcore_barrier()

        wid = _worker()
        crow0 = wid * NCH

        def fire(i, pp):
            jj = i - (i // KC) * KC

            @pl.when(i >= 2)
            def _():
                pltpu.make_async_copy(gbuf[pp], zsh.at[dstS[pp]],
                                      sems[pp]).wait()

            @pl.when(jj == 0)
            def _():
                row0 = pl.multiple_of(crow0 + i, 8)
                pltpu.sync_copy(src_hbm.at[pl.ds(row0, KC)], srcb)
                pltpu.sync_copy(dst_hbm.at[pl.ds(row0, KC)], dstb)
                if use_w:
                    pltpu.sync_copy(w_hbm.at[pl.ds(row0, KC)], wb)

            for g in range(CH // L):
                sl = pl.ds(g * L, L)
                srcS[pp][sl] = srcb[jj, sl]
                dstS[pp][sl] = dstb[jj, sl]
                if use_w:
                    wS[pp][sl] = wb[jj, sl]
            pltpu.async_copy(yp_hbm.at[srcS[pp]], gbuf[pp], sem[pp])

        def retire(qq):
            pltpu.make_async_copy(yp_hbm.at[pl.ds(0, CH)], gbuf[qq],
                                  sem[qq]).wait()
            if use_w:
                def scale(g, carry3):
                    wvec = wS[qq][pl.ds(g * L, L)]
                    for r in range(L):
                        wv = wvec[r]
                        e = g * L + r
                        for cc in range(D // L):
                            sl = pl.ds(cc * L, L)
                            gbuf[qq][e, sl] = gbuf[qq][e, sl] * wv
                    return carry3
                lax.fori_loop(0, CH // L, scale, 0)
            pltpu.async_copy(gbuf[qq], zsh.at[dstS[qq]], sems[qq], add=True)

        def piter(i, carry):
            p = i & 1

            @pl.when((i < NCH) & (p == 0))
            def _():
                fire(i, 0)

            @pl.when((i < NCH) & (p == 1))
            def _():
                fire(i, 1)

            @pl.when((i > 0) & (p == 1))
            def _():
                retire(0)

            @pl.when((i > 0) & (p == 0))
            def _():
                retire(1)
            return carry
        lax.fori_loop(0, NCH + 1, piter, 0)
        pltpu.make_async_copy(gbuf[0], zsh.at[dstS[0]], sems[0]).wait()
        pltpu.make_async_copy(gbuf[1], zsh.at[dstS[1]], sems[1]).wait()
        plsc.subcore_barrier()

        def drow(j, carry):
            r0 = s * RPT + j * ZR
            pltpu.sync_copy(zsh.at[pl.ds(r0, ZR)], zb)
            pltpu.sync_copy(zb, zp_hbm.at[pl.ds(c * NP + r0, ZR)])
            return carry
        lax.fori_loop(0, RPT // ZR, drow, 0)

    scratch = [
        pltpu.VMEM_SHARED((NP, D), f32),
        pltpu.VMEM((KC, CH), i32),
        pltpu.VMEM((KC, CH), i32),
        pltpu.VMEM((KC, CH), f32),
        pltpu.VMEM((CH, D), f32),
        pltpu.VMEM((CH, D), f32),
        pltpu.VMEM((CH,), i32),
        pltpu.VMEM((CH,), i32),
        pltpu.VMEM((CH,), i32),
        pltpu.VMEM((CH,), i32),
        pltpu.VMEM((CH,), f32),
        pltpu.VMEM((CH,), f32),
        pltpu.VMEM((64, D), f32),
        pltpu.SemaphoreType.DMA,
        pltpu.SemaphoreType.DMA,
        pltpu.SemaphoreType.DMA,
        pltpu.SemaphoreType.DMA,
    ]
    out_type = jax.ShapeDtypeStruct((NC * NP, D), f32)
    return pl.kernel(body, out_type=out_type, mesh=_mesh,
                     scratch_types=scratch, compiler_params=_cparams)


_step_a_w = _make_step_a(True)


def _step_b_body(zp_hbm, ycur_hbm, ct_hbm, n1_hbm,
                 ynew_hbm, ypnew_hbm,
                 z0b, z1b, ybuf, cbuf, obuf, opbuf, n1t):
    wid = _worker()
    r0 = wid * RPW
    pltpu.sync_copy(n1_hbm.at[pl.ds(r0, RPW)], n1t)

    RB = 32

    def brow(j, carry):
        rr = r0 + j * RB
        pltpu.sync_copy(zp_hbm.at[pl.ds(rr, RB)], z0b)
        pltpu.sync_copy(zp_hbm.at[pl.ds(NP + rr, RB)], z1b)
        pltpu.sync_copy(ycur_hbm.at[pl.ds(rr, RB)], ybuf)
        pltpu.sync_copy(ct_hbm.at[pl.ds(rr, RB)], cbuf)
        for h in range(RB // L):
            n1v = n1t[pl.ds(j * RB + h * L, L)]
            for r in range(L):
                nv = n1v[r]
                e = h * L + r
                for cc in range(D // L):
                    sl = pl.ds(cc * L, L)
                    out = 0.5 * ybuf[e, sl] \
                        + (0.5 * nv) * (z0b[e, sl] + z1b[e, sl]) \
                        + cbuf[e, sl]
                    obuf[e, sl] = out
                    opbuf[e, sl] = nv * out
        pltpu.sync_copy(obuf, ynew_hbm.at[pl.ds(rr, RB)])
        pltpu.sync_copy(opbuf, ypnew_hbm.at[pl.ds(rr, RB)])
        return carry
    lax.fori_loop(0, RPW // RB, brow, 0)


_step_b = pl.kernel(
    _step_b_body,
    out_type=(jax.ShapeDtypeStruct((NP, D), f32),
              jax.ShapeDtypeStruct((NP, D), f32)),
    mesh=_mesh,
    compiler_params=_cparams,
    scratch_types=[
        pltpu.VMEM((32, D), f32),
        pltpu.VMEM((32, D), f32),
        pltpu.VMEM((32, D), f32),
        pltpu.VMEM((32, D), f32),
        pltpu.VMEM((32, D), f32),
        pltpu.VMEM((32, D), f32),
        pltpu.VMEM((RPW,), f32),
    ])


def _make_prep_a(with_w):
    def body(*refs):
        if with_w:
            (src_hbm, dst_hbm, hn_hbm, pd_hbm,
             degp_hbm, w_hbm,
             degsh, degloc1, degloc, riB, srcb, dstb, wb, pdb, hnf,
             zbd) = refs
        else:
            (dst_hbm,
             degp_hbm,
             degsh, degloc1, degloc, riB, srcb, dstb, wb, pdb, hnf,
             zbd) = refs
        c = lax.axis_index("c")
        s = lax.axis_index("s")
        zv = _zeros16()
        it = lax.iota(i32, L)

        for j in range(8):
            for g in range(CH // L):
                zbd[j, pl.ds(g * L, L)] = zv

        @pl.when(s < NROW // 8)
        def _():
            pltpu.sync_copy(zbd, degsh.at[pl.ds(s * 8, 8)])

        def zloc(j, carry):
            degloc1[pl.ds(j * L, L)] = zv
            return carry
        lax.fori_loop(0, NP // L, zloc, 0)

        for g in range(NROW // L):
            riB[pl.ds(g * L, L)] = it + g * L
        plsc.subcore_barrier()

        if with_w:
            pltpu.sync_copy(hn_hbm.at[pl.ds(0, NP)], hnf)

        wid = _worker()
        bbase = wid * NB

        def block(i, carry):
            row0 = (bbase + i) * KC
            pltpu.sync_copy(dst_hbm.at[pl.ds(row0, KC)], dstb)
            if with_w:
                pltpu.sync_copy(src_hbm.at[pl.ds(row0, KC)], srcb)
                pltpu.sync_copy(pd_hbm.at[pl.ds(row0, KC)], pdb)

            def chunk(j, carry2):
                for g in range(CH // L):
                    sl = pl.ds(g * L, L)
                    dstv = dstb[j, sl]
                    if with_w:
                        srcv = srcb[j, sl]
                        hs = plsc.load_gather(hnf, [srcv])
                        hd = plsc.load_gather(hnf, [dstv])
                        wr = hs + hd - 2.0 * pdb[j, sl]
                        m = jnp.maximum(wr, 0.0) + 1e-7
                        wv = jnp.minimum(_rsqrt(m), 1.0 / TAU) + 1e-9
                        wb[j, sl] = wv
                    else:
                        wv = zv + 1.0
                    plsc.addupdate_scatter(degloc1, [dstv], wv)
                return carry2
            lax.fori_loop(0, KC, chunk, 0)
            if with_w:
                pltpu.sync_copy(wb, w_hbm.at[pl.ds(row0, KC)])
            return carry
        lax.fori_loop(0, NB, block, 0)

        def repack(j, carry):
            for g in range(CH // L):
                degloc[j, pl.ds(g * L, L)] = \
                    degloc1[pl.ds(j * CH + g * L, L)]
            return carry
        lax.fori_loop(0, NROW, repack, 0)

        pltpu.sync_copy(degloc, degsh.at[riB], add=True)
        plsc.subcore_barrier()

        @pl.when(s < NROW // 8)
        def _():
            pltpu.sync_copy(degsh.at[pl.ds(s * 8, 8)], zbd)
            pltpu.sync_copy(zbd, degp_hbm.at[pl.ds(c * NROW + s * 8, 8)])

    scratch = [
        pltpu.VMEM_SHARED((NROW, CH), f32),
        pltpu.VMEM((NP,), f32),
        pltpu.VMEM((NROW, CH), f32),
        pltpu.VMEM((NROW,), i32),
        pltpu.VMEM((KC, CH), i32),
        pltpu.VMEM((KC, CH), i32),
        pltpu.VMEM((KC, CH), f32),
        pltpu.VMEM((KC, CH), f32),
        pltpu.VMEM((NP,), f32),
        pltpu.VMEM((8, CH), f32),
    ]
    outs = [jax.ShapeDtypeStruct((NC * NROW, CH), f32)]
    if with_w:
        outs.append(jax.ShapeDtypeStruct((EP // CH, CH), f32))
    out_type = outs[0] if not with_w else tuple(outs)
    return pl.kernel(body, out_type=out_type, mesh=_mesh,
                     scratch_types=scratch, compiler_params=_cparams)


_prep_a1 = _make_prep_a(False)
_prep_a2 = _make_prep_a(True)


def _prep_b_body(x_hbm, ysrc_hbm, degp_hbm,
                 n1_hbm, ct_hbm, yp_hbm,
                 degb, xbuf, ybuf, ctb, ypb, n1t):
    s = lax.axis_index("s")

    pltpu.sync_copy(degp_hbm, degb)

    RB = 32

    def nrow(j, carry):
        r0 = s * RPT + j * RB
        pltpu.sync_copy(x_hbm.at[pl.ds(r0, RB)], xbuf)
        pltpu.sync_copy(ysrc_hbm.at[pl.ds(r0, RB)], ybuf)
        for h in range(RB // L):
            jj = j * (RB // L) + h
            dr = s * (NROW // NS) + (jj >> 3)
            dsl = pl.ds((jj & 7) * L, L)
            degv = degb[dr, dsl] + degb[NROW + dr, dsl]
            n1v = _rsqrt(degv)
            dvv = 0.5 * (n1v * n1v)
            n1t[pl.ds(jj * L, L)] = n1v
            for r in range(L):
                nv = n1v[r]
                dv = dvv[r]
                e = h * L + r
                for cc in range(D // L):
                    sl = pl.ds(cc * L, L)
                    ctb[e, sl] = dv * xbuf[e, sl]
                    ypb[e, sl] = nv * ybuf[e, sl]
        pltpu.sync_copy(ctb, ct_hbm.at[pl.ds(r0, RB)])
        pltpu.sync_copy(ypb, yp_hbm.at[pl.ds(r0, RB)])
        return carry
    lax.fori_loop(0, RPT // RB, nrow, 0)
    pltpu.sync_copy(n1t, n1_hbm.at[pl.ds(s * RPT, RPT)])


_prep_b = pl.kernel(
    _prep_b_body,
    out_type=(jax.ShapeDtypeStruct((NP,), f32),
              jax.ShapeDtypeStruct((NP, D), f32),
              jax.ShapeDtypeStruct((NP, D), f32)),
    mesh=_mesh,
    compiler_params=_cparams,
    scratch_types=[
        pltpu.VMEM((NC * NROW, CH), f32),
        pltpu.VMEM((32, D), f32),
        pltpu.VMEM((32, D), f32),
        pltpu.VMEM((32, D), f32),
        pltpu.VMEM((32, D), f32),
        pltpu.VMEM((RPT,), f32),
    ])


def _lane_sum(tbuf, it):
    acc = _zeros16()
    for l in range(L):
        col = jnp.zeros((L,), i32) + l
        acc = acc + plsc.load_gather(tbuf, [it, col])
    return acc


def _attn_body(ys_hbm, src_hbm, dst_hbm, hn_hbm, pd_hbm,
               srcb, dstb, ga0, ga1, gb0, gb1, srcS0, srcS1, dstS0, dstS1,
               ybuf, tbuf, hnt, pdb, sma0, sma1, smb0, smb1):
    it = lax.iota(i32, L)
    wid = _worker()

    r0 = wid * RPW

    def hrow(j, carry):
        pltpu.sync_copy(ys_hbm.at[pl.ds(r0 + j * L, L)], ybuf)
        for r in range(L):
            acc = _zeros16()
            for sl in range(D // L):
                v = ybuf[r, pl.ds(sl * L, L)]
                acc = acc + v * v
            tbuf[r, pl.ds(0, L)] = acc
        hnt[pl.ds(j * L, L)] = _lane_sum(tbuf, it)
        return carry
    lax.fori_loop(0, RPW // L, hrow, 0)
    pltpu.sync_copy(hnt, hn_hbm.at[pl.ds(r0, RPW)])

    crow0 = wid * NCH
    ga = (ga0, ga1)
    gb = (gb0, gb1)
    srcS = (srcS0, srcS1)
    dstS = (dstS0, dstS1)
    sma = (sma0, sma1)
    smb = (smb0, smb1)

    def fire(i, pp):
        jj = i - (i // KC) * KC

        @pl.when(jj == 0)
        def _():
            row0 = pl.multiple_of(crow0 + i, 8)
            pltpu.sync_copy(src_hbm.at[pl.ds(row0, KC)], srcb)
            pltpu.sync_copy(dst_hbm.at[pl.ds(row0, KC)], dstb)
        for g in range(CH // L):
            sl = pl.ds(g * L, L)
            srcS[pp][sl] = srcb[jj, sl]
            dstS[pp][sl] = dstb[jj, sl]
        pltpu.async_copy(ys_hbm.at[srcS[pp]], ga[pp], sma[pp])
        pltpu.async_copy(ys_hbm.at[dstS[pp]], gb[pp], smb[pp])

    def retire(i, qq):
        iq = i - 1
        jq = iq - (iq // KC) * KC
        pltpu.make_async_copy(ys_hbm.at[pl.ds(0, CH)], ga[qq],
                              sma[qq]).wait()
        pltpu.make_async_copy(ys_hbm.at[pl.ds(0, CH)], gb[qq],
                              smb[qq]).wait()

        def grp(g, carry3):
            for r in range(L):
                e = g * L + r
                acc = _zeros16()
                for sl in range(D // L):
                    csl = pl.ds(sl * L, L)
                    acc = acc + ga[qq][e, csl] * gb[qq][e, csl]
                tbuf[r, pl.ds(0, L)] = acc
            pdb[jq, pl.ds(g * L, L)] = _lane_sum(tbuf, it)
            return carry3
        lax.fori_loop(0, CH // L, grp, 0)

        @pl.when(jq == KC - 1)
        def _():
            row0 = pl.multiple_of(crow0 + iq - (KC - 1), 8)
            pltpu.sync_copy(pdb, pd_hbm.at[pl.ds(row0, KC)])

    def piter(i, carry):
        p = i & 1

        @pl.when((i < NCH) & (p == 0))
        def _():
            fire(i, 0)

        @pl.when((i < NCH) & (p == 1))
        def _():
            fire(i, 1)

        @pl.when((i > 0) & (p == 1))
        def _():
            retire(i, 0)

        @pl.when((i > 0) & (p == 0))
        def _():
            retire(i, 1)
        return carry
    lax.fori_loop(0, NCH + 1, piter, 0)


_attn = pl.kernel(
    _attn_body,
    out_type=(jax.ShapeDtypeStruct((NP,), f32),
              jax.ShapeDtypeStruct((EP // CH, CH), f32)),
    mesh=_mesh,
    compiler_params=_cparams,
    scratch_types=[
        pltpu.VMEM((KC, CH), i32),
        pltpu.VMEM((KC, CH), i32),
        pltpu.VMEM((CH, D), f32),
        pltpu.VMEM((CH, D), f32),
        pltpu.VMEM((CH, D), f32),
        pltpu.VMEM((CH, D), f32),
        pltpu.VMEM((CH,), i32),
        pltpu.VMEM((CH,), i32),
        pltpu.VMEM((CH,), i32),
        pltpu.VMEM((CH,), i32),
        pltpu.VMEM((L, D), f32),
        pltpu.VMEM((L, L), f32),
        pltpu.VMEM((RPW,), f32),
        pltpu.VMEM((KC, CH), f32),
        pltpu.SemaphoreType.DMA,
        pltpu.SemaphoreType.DMA,
        pltpu.SemaphoreType.DMA,
        pltpu.SemaphoreType.DMA,
    ])


def kernel(x, edge_index):
    src = edge_index[0].astype(i32)
    dst = edge_index[1].astype(i32)
    srcp = jnp.zeros((EP,), i32).at[:E].set(src).reshape(EP // CH, CH)
    dstp = jnp.full((EP,), NP - 1, i32).at[:E].set(dst).reshape(EP // CH, CH)

    xs = jnp.zeros((NP, D), f32).at[:N].set(x)

    degp = _prep_a1(dstp)
    n1, ct, yp = _prep_b(xs, xs, degp)
    ys = xs
    w1 = jnp.ones((EP // CH, CH), f32)
    for _ in range(4):
        zp = _step_a_w(yp, srcp, dstp, w1)
        ys, yp = _step_b(zp, ys, ct, n1)
    hn, pd = _attn(ys, srcp, dstp)
    degp, w = _prep_a2(srcp, dstp, hn, pd)
    n1, ct, yp = _prep_b(xs, ys, degp)
    for _ in range(4):
        zp = _step_a_w(yp, srcp, dstp, w)
        ys, yp = _step_b(zp, ys, ct, n1)
    return ys[:N]
